# baseline (speedup 1.0000x reference)
import jax
import jax.numpy as jnp
from jax import lax
from jax.experimental import pallas as pl
from jax.experimental.pallas import tpu as pltpu
from jax.experimental.pallas import tpu_sc as plsc

DIM = 32
BATCH = 16384
NUM_CORES = 2
NUM_SUBCORES = 16
NUM_WORKERS = NUM_CORES * NUM_SUBCORES
RANGE = 32768
CHU = 512
NCH = RANGE // CHU
USERS = 1000000
WIDE = 128
LCAP = 1024
CCAP = 64
TRASH0 = BATCH
GROWS = BATCH + NUM_WORKERS * CCAP


def _scan_body(uidx_hbm, pidx_hbm, nidx_hbm, ut_hbm, it_hbm,
               tut_hbm, tit_hbm,
               gu_hbm, gi_hbm, gj_hbm,
               idxbuf, sb0, sb1, sbt,
               lrel_u, lb_u, lrel_p, lb_p, lrel_n, lb_n,
               crel, cb2d, stage, sem0, sem1, ssem):
    wid = lax.axis_index("s") * NUM_CORES + lax.axis_index("c")
    lo = wid * RANGE
    lane = lax.iota(jnp.int32, 16)
    trash = TRASH0 + wid * CCAP

    n_full = jnp.where(wid < 30, NCH, jnp.where(wid == 30, 33, 0))

    def extract(idx_hbm, lrel, lb):
        pltpu.sync_copy(idx_hbm, idxbuf)

        def body(i, fill):
            v = idxbuf[pl.ds(i * 16, 16)]
            rel = v - lo
            m = (rel >= 0) & (rel < RANGE)
            ci = plsc.cumsum(m.astype(jnp.int32))
            slot = fill + ci - 1
            plsc.store_scatter(lrel, [slot], rel, mask=m)
            plsc.store_scatter(lb, [slot], lane + i * 16, mask=m)
            return fill + ci[15]

        return lax.fori_loop(0, BATCH // 16, body, 0)

    def process(c, buf, lists, width=CHU):
        clo = c * CHU
        for lrel, lb, fill, g_hbm in lists:
            for g in range(CCAP // 16):
                crel[pl.ds(g * 16, 16)] = jnp.zeros((16,), jnp.int32)
                cb2d[g, :] = trash + g * 16 + lane

            nv = (fill + 15) >> 4

            def fbody(v, cc):
                rel = lrel[pl.ds(v * 16, 16)]
                b = lb[pl.ds(v * 16, 16)]
                m = (rel >= clo) & (rel < clo + width)
                ci = plsc.cumsum(m.astype(jnp.int32))
                slot = cc + ci - 1
                plsc.store_scatter(crel, [slot], rel - clo, mask=m)
                plsc.store_scatter(cb2d, [slot >> 4, slot & 15], b, mask=m)
                return cc + ci[15]

            ccnt = lax.fori_loop(0, nv, fbody, 0)
            ng = (ccnt + 15) >> 4

            def gbody(g, _):
                rel16 = plsc.load_gather(crel, [g * 16 + lane])
                for d in range(DIM):
                    dv = jnp.full((16,), d, jnp.int32)
                    val = plsc.load_gather(buf, [dv, rel16])
                    plsc.store_scatter(stage, [g * 16 + lane, dv], val)
                return 0

            lax.fori_loop(0, ng, gbody, 0)

            def sbody(g, _):
                o16 = pl.multiple_of(g * 16, 16)
                pltpu.async_copy(
                    stage.at[pl.ds(o16, 16)],
                    g_hbm.at[cb2d.at[g]], ssem)
                pltpu.make_async_copy(
                    stage.at[pl.ds(o16, 16)],
                    g_hbm.at[cb2d.at[g]], ssem).wait()
                return 0

            lax.fori_loop(0, ng, sbody, 0)

    def scan(tab, lists, n_chunks):
        def fire(c, buf, sem):
            off = pl.multiple_of(lo + c * CHU, 128)
            return pltpu.async_copy(tab.at[:, pl.ds(off, CHU)], buf, sem)

        def drain(buf, sem):
            pltpu.make_async_copy(tab.at[:, pl.ds(lo, CHU)], buf, sem).wait()

        @pl.when(n_chunks > 0)
        def _():
            fire(0, sb0, sem0)

            @pl.when(n_chunks > 1)
            def _():
                fire(1, sb1, sem1)

            def cbody(t, _):
                c0 = t * 2
                drain(sb0, sem0)
                process(c0, sb0, lists)

                @pl.when(c0 + 2 < n_chunks)
                def _():
                    fire(c0 + 2, sb0, sem0)

                @pl.when(c0 + 1 < n_chunks)
                def _():
                    drain(sb1, sem1)
                    process(c0 + 1, sb1, lists)

                    @pl.when(c0 + 3 < n_chunks)
                    def _():
                        fire(c0 + 3, sb1, sem1)
                return 0

            lax.fori_loop(0, (n_chunks + 1) // 2, cbody, 0)

    fill_u = extract(uidx_hbm, lrel_u, lb_u)
    scan(ut_hbm, [(lrel_u, lb_u, fill_u, gu_hbm)], n_full)

    fill_p = extract(pidx_hbm, lrel_p, lb_p)
    fill_n = extract(nidx_hbm, lrel_n, lb_n)
    scan(it_hbm, [(lrel_p, lb_p, fill_p, gi_hbm),
                  (lrel_n, lb_n, fill_n, gj_hbm)], n_full)

    @pl.when(wid == 30)
    def _():
        pltpu.sync_copy(tut_hbm, sbt)
        process(33, sbt, [(lrel_u, lb_u, fill_u, gu_hbm)], width=64)
        pltpu.sync_copy(tit_hbm, sbt)
        process(33, sbt, [(lrel_p, lb_p, fill_p, gi_hbm),
                          (lrel_n, lb_n, fill_n, gj_hbm)], width=64)


def _dot_block(u_ref, i_ref, j_ref, p_ref, n_ref):
    u = u_ref[...]
    mask = lax.broadcasted_iota(jnp.int32, u.shape, 1) < DIM
    ui = jnp.where(mask, u * i_ref[...], 0.0)
    uj = jnp.where(mask, u * j_ref[...], 0.0)
    p_ref[...] = jnp.sum(ui, axis=1, keepdims=True)
    n_ref[...] = jnp.sum(uj, axis=1, keepdims=True)


@jax.jit
def _bpr(batch_user, batch_pos_item, batch_neg_item, user_emb, item_emb):
    u_t = user_emb.T
    i_t = item_emb.T
    tail0 = 30 * RANGE + 33 * CHU
    t_u = lax.slice(u_t, (0, tail0), (DIM, USERS))
    t_i = lax.slice(i_t, (0, tail0), (DIM, USERS))
    mesh = plsc.VectorSubcoreMesh(core_axis_name="c", subcore_axis_name="s")
    kfn = pl.kernel(
        _scan_body,
        out_type=(
            jax.ShapeDtypeStruct((GROWS, WIDE), jnp.float32),
            jax.ShapeDtypeStruct((GROWS, WIDE), jnp.float32),
            jax.ShapeDtypeStruct((GROWS, WIDE), jnp.float32),
        ),
        mesh=mesh,
        compiler_params=pltpu.CompilerParams(needs_layout_passes=False),
        scratch_types=[
            pltpu.VMEM((BATCH,), jnp.int32),
            pltpu.VMEM((DIM, CHU), jnp.float32),
            pltpu.VMEM((DIM, CHU), jnp.float32),
            pltpu.VMEM((DIM, 64), jnp.float32),
            pltpu.VMEM((LCAP,), jnp.int32),
            pltpu.VMEM((LCAP,), jnp.int32),
            pltpu.VMEM((LCAP,), jnp.int32),
            pltpu.VMEM((LCAP,), jnp.int32),
            pltpu.VMEM((LCAP,), jnp.int32),
            pltpu.VMEM((LCAP,), jnp.int32),
            pltpu.VMEM((CCAP,), jnp.int32),
            pltpu.VMEM((CCAP // 16, 16), jnp.int32),
            pltpu.VMEM((CCAP, WIDE), jnp.float32),
            pltpu.SemaphoreType.DMA,
            pltpu.SemaphoreType.DMA,
            pltpu.SemaphoreType.DMA,
        ],
    )
    gu, gi, gj = kfn(batch_user, batch_pos_item, batch_neg_item, u_t, i_t,
                     t_u, t_i)

    grid = 8
    blk = BATCH // grid
    pos, neg = pl.pallas_call(
        _dot_block,
        grid=(grid,),
        in_specs=[
            pl.BlockSpec((blk, WIDE), lambda i: (i, 0)),
            pl.BlockSpec((blk, WIDE), lambda i: (i, 0)),
            pl.BlockSpec((blk, WIDE), lambda i: (i, 0)),
        ],
        out_specs=[
            pl.BlockSpec((blk, 1), lambda i: (i, 0)),
            pl.BlockSpec((blk, 1), lambda i: (i, 0)),
        ],
        out_shape=[
            jax.ShapeDtypeStruct((BATCH, 1), jnp.float32),
            jax.ShapeDtypeStruct((BATCH, 1), jnp.float32),
        ],
    )(gu, gi, gj)
    return pos, neg


def kernel(batch_user, batch_pos_item, batch_neg_item, user_emb, item_emb):
    return _bpr(batch_user, batch_pos_item, batch_neg_item,
                user_emb, item_emb)


# speedup vs baseline: 1.3191x; 1.3191x over previous
import jax
import jax.numpy as jnp
from jax import lax
from jax.experimental import pallas as pl
from jax.experimental.pallas import tpu as pltpu
from jax.experimental.pallas import tpu_sc as plsc

DIM = 32
BATCH = 16384
NUM_CORES = 2
NUM_SUBCORES = 16
NUM_WORKERS = NUM_CORES * NUM_SUBCORES
RANGE = 32768
CHU = 1024
NCH = RANGE // CHU
USERS = 1000000
WIDE = 128
LCAP = 1024
CCAP = 64
TRASH0 = BATCH
GROWS = BATCH + NUM_WORKERS * CCAP
W30LO = 30 * RANGE
W30FULL = 16
TAIL512 = W30LO + W30FULL * CHU
TAIL64 = TAIL512 + 512


def _scan_body(uidx_hbm, pidx_hbm, nidx_hbm, ut_hbm, it_hbm,
               tut_hbm, tit_hbm,
               gu_hbm, gi_hbm, gj_hbm,
               idxbuf, sb0, sb1, sbt5, sbt,
               lrel_u, lb_u, lrel_p, lb_p, lrel_n, lb_n,
               crel, cb2d_a, cb2d_b, stage_a, stage_b, sem0, sem1,
               ssem_a, ssem_b):
    wid = lax.axis_index("s") * NUM_CORES + lax.axis_index("c")
    lo = wid * RANGE
    lane = lax.iota(jnp.int32, 16)
    trash = TRASH0 + wid * CCAP

    n_full = jnp.where(wid < 30, NCH, jnp.where(wid == 30, W30FULL, 0))

    def extract(idx_hbm, lrel, lb):
        pltpu.sync_copy(idx_hbm, idxbuf)

        def body(i, fill):
            v = idxbuf[pl.ds(i * 16, 16)]
            rel = v - lo
            m = (rel >= 0) & (rel < RANGE)
            ci = plsc.cumsum(m.astype(jnp.int32))
            slot = fill + ci - 1
            plsc.store_scatter(lrel, [slot], rel, mask=m)
            plsc.store_scatter(lb, [slot], lane + i * 16, mask=m)
            return fill + plsc.all_reduce_population_count(m)[0]

        return lax.fori_loop(0, BATCH // 16, body, 0)

    def drain_scatters(stage, g_hbm, ng, ssem, cb2d):
        def dbody(g, _):
            o16 = pl.multiple_of(g * 16, 16)
            pltpu.make_async_copy(
                stage.at[pl.ds(o16, 16)],
                g_hbm.at[cb2d.at[g]], ssem).wait()
            return 0
        lax.fori_loop(0, ng, dbody, 0)

    def process(clo, buf, lists, width, prev_ngs):
        ngs = []
        for (lrel, lb, fill, g_hbm, stage, ssem, cb2d), png in zip(
                lists, prev_ngs):
            drain_scatters(stage, g_hbm, png, ssem, cb2d)

            for g in range(CCAP // 16):
                crel[pl.ds(g * 16, 16)] = jnp.zeros((16,), jnp.int32)
                cb2d[g, :] = trash + g * 16 + lane

            nv = (fill + 15) >> 4

            def fbody(v, cc, lrel=lrel, lb=lb, cb2d=cb2d):
                rel = lrel[pl.ds(v * 16, 16)]
                b = lb[pl.ds(v * 16, 16)]
                m = (rel >= clo) & (rel < clo + width)
                ci = plsc.cumsum(m.astype(jnp.int32))
                slot = cc + ci - 1
                plsc.store_scatter(crel, [slot], rel - clo, mask=m)
                plsc.store_scatter(cb2d, [slot >> 4, slot & 15], b, mask=m)
                return cc + plsc.all_reduce_population_count(m)[0]

            ccnt = lax.fori_loop(0, nv, fbody, 0)
            ng = (ccnt + 15) >> 4

            def gather_group(g, buf=buf, stage=stage):
                rel16 = plsc.load_gather(crel, [g * 16 + lane])
                for d in range(DIM):
                    dv = jnp.full((16,), d, jnp.int32)
                    val = plsc.load_gather(buf, [dv, rel16])
                    plsc.store_scatter(stage, [g * 16 + lane, dv], val)

            @pl.when(ng > 0)
            def _(gather_group=gather_group):
                gather_group(0)

            def sgbody(g, _, g_hbm=g_hbm, stage=stage, ssem=ssem,
                       cb2d=cb2d, gather_group=gather_group, ng=ng):
                @pl.when(g + 1 < ng)
                def _():
                    gather_group(g + 1)
                o16 = pl.multiple_of(g * 16, 16)
                pltpu.async_copy(
                    stage.at[pl.ds(o16, 16)],
                    g_hbm.at[cb2d.at[g]], ssem).wait()
                return 0

            lax.fori_loop(0, ng, sgbody, 0)
            ngs.append(jnp.int32(0))
        return tuple(ngs)

    def fire(tab, c, buf, sem):
        off = pl.multiple_of(lo + c * CHU, 128)
        pltpu.async_copy(tab.at[:, pl.ds(off, CHU)], buf, sem)

    def drain_chunk(tab, buf, sem):
        pltpu.make_async_copy(tab.at[:, pl.ds(lo, CHU)], buf, sem).wait()

    def scan(tab, lists, n_chunks, init_ngs):
        def cbody(t, ngs):
            c0 = t * 2
            drain_chunk(tab, sb0, sem0)
            ngs = process(c0 * CHU, sb0, lists, CHU, ngs)

            @pl.when(c0 + 2 < n_chunks)
            def _():
                fire(tab, c0 + 2, sb0, sem0)

            drain_chunk(tab, sb1, sem1)
            ngs = process((c0 + 1) * CHU, sb1, lists, CHU, ngs)

            @pl.when(c0 + 3 < n_chunks)
            def _():
                fire(tab, c0 + 3, sb1, sem1)
            return ngs

        return lax.fori_loop(0, n_chunks // 2, cbody, init_ngs)

    zero = jnp.int32(0)

    @pl.when(n_full > 0)
    def _():
        fire(ut_hbm, 0, sb0, sem0)
        fire(ut_hbm, 1, sb1, sem1)

    fill_u = extract(uidx_hbm, lrel_u, lb_u)
    u_lists = [(lrel_u, lb_u, fill_u, gu_hbm, stage_a, ssem_a, cb2d_a)]
    ngs_u = scan(ut_hbm, u_lists, n_full, (zero,))

    @pl.when(wid == 30)
    def _():
        pltpu.sync_copy(ut_hbm.at[:, pl.ds(TAIL512 - W30LO + lo, 512)],
                        sbt5)
        n1 = process(TAIL512 - W30LO, sbt5, u_lists, 512, ngs_u)
        pltpu.sync_copy(tut_hbm, sbt)
        n2 = process(TAIL64 - W30LO, sbt, u_lists, 64, n1)
        drain_scatters(stage_a, gu_hbm, n2[0], ssem_a, cb2d_a)

    @pl.when(wid != 30)
    def _():
        drain_scatters(stage_a, gu_hbm, ngs_u[0], ssem_a, cb2d_a)

    @pl.when(n_full > 0)
    def _():
        fire(it_hbm, 0, sb0, sem0)
        fire(it_hbm, 1, sb1, sem1)

    fill_p = extract(pidx_hbm, lrel_p, lb_p)
    fill_n = extract(nidx_hbm, lrel_n, lb_n)
    i_lists = [(lrel_p, lb_p, fill_p, gi_hbm, stage_a, ssem_a, cb2d_a),
               (lrel_n, lb_n, fill_n, gj_hbm, stage_b, ssem_b, cb2d_b)]
    ngs_i = scan(it_hbm, i_lists, n_full, (zero, zero))

    @pl.when(wid == 30)
    def _():
        pltpu.sync_copy(it_hbm.at[:, pl.ds(TAIL512 - W30LO + lo, 512)],
                        sbt5)
        n1 = process(TAIL512 - W30LO, sbt5, i_lists, 512, ngs_i)
        pltpu.sync_copy(tit_hbm, sbt)
        n2 = process(TAIL64 - W30LO, sbt, i_lists, 64, n1)
        drain_scatters(stage_a, gi_hbm, n2[0], ssem_a, cb2d_a)
        drain_scatters(stage_b, gj_hbm, n2[1], ssem_b, cb2d_b)

    @pl.when(wid != 30)
    def _():
        drain_scatters(stage_a, gi_hbm, ngs_i[0], ssem_a, cb2d_a)
        drain_scatters(stage_b, gj_hbm, ngs_i[1], ssem_b, cb2d_b)


def _dot_block(u_ref, i_ref, j_ref, p_ref, n_ref):
    u = u_ref[...]
    mask = lax.broadcasted_iota(jnp.int32, u.shape, 1) < DIM
    ui = jnp.where(mask, u * i_ref[...], 0.0)
    uj = jnp.where(mask, u * j_ref[...], 0.0)
    p_ref[...] = jnp.sum(ui, axis=1, keepdims=True)
    n_ref[...] = jnp.sum(uj, axis=1, keepdims=True)


@jax.jit
def _bpr(batch_user, batch_pos_item, batch_neg_item, user_emb, item_emb):
    u_t = user_emb.T
    i_t = item_emb.T
    t_u = lax.slice(u_t, (0, TAIL64), (DIM, USERS))
    t_i = lax.slice(i_t, (0, TAIL64), (DIM, USERS))
    mesh = plsc.VectorSubcoreMesh(core_axis_name="c", subcore_axis_name="s")
    kfn = pl.kernel(
        _scan_body,
        out_type=(
            jax.ShapeDtypeStruct((GROWS, WIDE), jnp.float32),
            jax.ShapeDtypeStruct((GROWS, WIDE), jnp.float32),
            jax.ShapeDtypeStruct((GROWS, WIDE), jnp.float32),
        ),
        mesh=mesh,
        compiler_params=pltpu.CompilerParams(needs_layout_passes=False),
        scratch_types=[
            pltpu.VMEM((BATCH,), jnp.int32),
            pltpu.VMEM((DIM, CHU), jnp.float32),
            pltpu.VMEM((DIM, CHU), jnp.float32),
            pltpu.VMEM((DIM, 512), jnp.float32),
            pltpu.VMEM((DIM, 64), jnp.float32),
            pltpu.VMEM((LCAP,), jnp.int32),
            pltpu.VMEM((LCAP,), jnp.int32),
            pltpu.VMEM((LCAP,), jnp.int32),
            pltpu.VMEM((LCAP,), jnp.int32),
            pltpu.VMEM((LCAP,), jnp.int32),
            pltpu.VMEM((LCAP,), jnp.int32),
            pltpu.VMEM((CCAP,), jnp.int32),
            pltpu.VMEM((CCAP // 16, 16), jnp.int32),
            pltpu.VMEM((CCAP // 16, 16), jnp.int32),
            pltpu.VMEM((CCAP, WIDE), jnp.float32),
            pltpu.VMEM((CCAP, WIDE), jnp.float32),
            pltpu.SemaphoreType.DMA,
            pltpu.SemaphoreType.DMA,
            pltpu.SemaphoreType.DMA,
            pltpu.SemaphoreType.DMA,
        ],
    )
    gu, gi, gj = kfn(batch_user, batch_pos_item, batch_neg_item, u_t, i_t,
                     t_u, t_i)

    grid = 8
    blk = BATCH // grid
    pos, neg = pl.pallas_call(
        _dot_block,
        grid=(grid,),
        in_specs=[
            pl.BlockSpec((blk, WIDE), lambda i: (i, 0)),
            pl.BlockSpec((blk, WIDE), lambda i: (i, 0)),
            pl.BlockSpec((blk, WIDE), lambda i: (i, 0)),
        ],
        out_specs=[
            pl.BlockSpec((blk, 1), lambda i: (i, 0)),
            pl.BlockSpec((blk, 1), lambda i: (i, 0)),
        ],
        out_shape=[
            jax.ShapeDtypeStruct((BATCH, 1), jnp.float32),
            jax.ShapeDtypeStruct((BATCH, 1), jnp.float32),
        ],
    )(gu, gi, gj)
    return pos, neg


def kernel(batch_user, batch_pos_item, batch_neg_item, user_emb, item_emb):
    return _bpr(batch_user, batch_pos_item, batch_neg_item,
                user_emb, item_emb)


# speedup vs baseline: 1.3798x; 1.0460x over previous
import jax
import jax.numpy as jnp
from jax import lax
from jax.experimental import pallas as pl
from jax.experimental.pallas import tpu as pltpu
from jax.experimental.pallas import tpu_sc as plsc

DIM = 32
BATCH = 16384
NUM_CORES = 2
NUM_SUBCORES = 16
NUM_WORKERS = NUM_CORES * NUM_SUBCORES
RANGE = 32768
CHU = 1024
NCH = RANGE // CHU
USERS = 1000000
WIDE = 128
LCAP = 1024
CCAP = 64
TRASH0 = BATCH
GROWS = BATCH + NUM_WORKERS * CCAP
W30LO = 30 * RANGE
W30FULL = 16
TAIL512 = W30LO + W30FULL * CHU
TAIL64 = TAIL512 + 512


def _scan_body(uidx_hbm, pidx_hbm, nidx_hbm, ut_hbm, it_hbm,
               tut_hbm, tit_hbm,
               gu_hbm, gi_hbm, gj_hbm,
               idxbuf, sb0, sb1, sbt,
               lrel_u, lb_u, lrel_p, lb_p, lrel_n, lb_n,
               crel, cb2d_a, cb2d_b, cb2d_c, cb2d_d,
               stage_a, stage_b, stage_c, stage_d, sem0, sem1,
               ssem_a, ssem_b, ssem_c, ssem_d):
    wid = lax.axis_index("s") * NUM_CORES + lax.axis_index("c")
    lo = wid * RANGE
    lane = lax.iota(jnp.int32, 16)
    trash = TRASH0 + wid * CCAP

    n_full = jnp.where(wid < 30, NCH, jnp.where(wid == 30, W30FULL, 0))

    def extract(idx_hbm, lrel, lb):
        pltpu.sync_copy(idx_hbm, idxbuf)

        def body(i, fill):
            v = idxbuf[pl.ds(i * 16, 16)]
            rel = v - lo
            m = (rel >= 0) & (rel < RANGE)
            ci = plsc.cumsum(m.astype(jnp.int32))
            slot = fill + ci - 1
            plsc.store_scatter(lrel, [slot], rel, mask=m)
            plsc.store_scatter(lb, [slot], lane + i * 16, mask=m)
            return fill + plsc.all_reduce_population_count(m)[0]

        return lax.fori_loop(0, BATCH // 16, body, 0)

    def drain_scatters(stage, g_hbm, ng, ssem, cb2d):
        def dbody(g, _):
            o16 = pl.multiple_of(g * 16, 16)
            pltpu.make_async_copy(
                stage.at[pl.ds(o16, 16)],
                g_hbm.at[cb2d.at[g]], ssem).wait()
            return 0
        lax.fori_loop(0, ng, dbody, 0)

    def process(clo, buf, lists, width, prev_ngs, par):
        ngs = []
        for (lrel, lb, fill, g_hbm, stages, ssems, cb2ds), pngs in zip(
                lists, prev_ngs):
            stage = stages[par]
            ssem = ssems[par]
            cb2d = cb2ds[par]
            png = pngs[par]
            drain_scatters(stage, g_hbm, png, ssem, cb2d)

            for g in range(CCAP // 16):
                crel[pl.ds(g * 16, 16)] = jnp.zeros((16,), jnp.int32)
                cb2d[g, :] = trash + g * 16 + lane

            nv = (fill + 15) >> 4

            def fbody(v, cc, lrel=lrel, lb=lb, cb2d=cb2d):
                rel = lrel[pl.ds(v * 16, 16)]
                b = lb[pl.ds(v * 16, 16)]
                m = (rel >= clo) & (rel < clo + width)
                ci = plsc.cumsum(m.astype(jnp.int32))
                slot = cc + ci - 1
                plsc.store_scatter(crel, [slot], rel - clo, mask=m)
                plsc.store_scatter(cb2d, [slot >> 4, slot & 15], b, mask=m)
                return cc + plsc.all_reduce_population_count(m)[0]

            ccnt = lax.fori_loop(0, nv, fbody, 0)
            ng = (ccnt + 15) >> 4

            def gather_group(g, buf=buf, stage=stage):
                rel16 = plsc.load_gather(crel, [g * 16 + lane])
                for d in range(DIM):
                    dv = jnp.full((16,), d, jnp.int32)
                    val = plsc.load_gather(buf, [dv, rel16])
                    plsc.store_scatter(stage, [g * 16 + lane, dv], val)

            @pl.when(ng > 0)
            def _(gather_group=gather_group):
                gather_group(0)

            def sgbody(g, _, g_hbm=g_hbm, stage=stage, ssem=ssem,
                       cb2d=cb2d, gather_group=gather_group, ng=ng):
                @pl.when(g + 1 < ng)
                def _():
                    gather_group(g + 1)
                o16 = pl.multiple_of(g * 16, 16)
                pltpu.async_copy(
                    stage.at[pl.ds(o16, 16)],
                    g_hbm.at[cb2d.at[g]], ssem)
                return 0

            lax.fori_loop(0, ng, sgbody, 0)
            ngs.append(tuple(ng if q == par else pngs[q] for q in (0, 1)))
        return tuple(ngs)

    def fire(tab, c, buf, sem):
        off = pl.multiple_of(lo + c * CHU, 128)
        pltpu.async_copy(tab.at[:, pl.ds(off, CHU)], buf, sem)

    def drain_chunk(tab, buf, sem):
        pltpu.make_async_copy(tab.at[:, pl.ds(lo, CHU)], buf, sem).wait()

    def scan(tab, lists, n_chunks, init_ngs):
        def cbody(t, ngs):
            c0 = t * 2
            drain_chunk(tab, sb0, sem0)
            ngs = process(c0 * CHU, sb0, lists, CHU, ngs, 0)

            @pl.when(c0 + 2 < n_chunks)
            def _():
                fire(tab, c0 + 2, sb0, sem0)

            drain_chunk(tab, sb1, sem1)
            ngs = process((c0 + 1) * CHU, sb1, lists, CHU, ngs, 1)

            @pl.when(c0 + 3 < n_chunks)
            def _():
                fire(tab, c0 + 3, sb1, sem1)
            return ngs

        return lax.fori_loop(0, n_chunks // 2, cbody, init_ngs)

    zero = jnp.int32(0)

    @pl.when(n_full > 0)
    def _():
        fire(ut_hbm, 0, sb0, sem0)
        fire(ut_hbm, 1, sb1, sem1)

    fill_u = extract(uidx_hbm, lrel_u, lb_u)
    u_lists = [(lrel_u, lb_u, fill_u, gu_hbm, (stage_a, stage_b),
                (ssem_a, ssem_b), (cb2d_a, cb2d_b))]
    zz = (zero, zero)
    ngs_u = scan(ut_hbm, u_lists, n_full, ((zero, zero),))

    @pl.when(wid == 30)
    def _():
        pltpu.sync_copy(ut_hbm.at[:, pl.ds(TAIL512 - W30LO + lo, 512)],
                        sb0.at[:, pl.ds(0, 512)])
        n1 = process(TAIL512 - W30LO, sb0, u_lists, 512, ngs_u, 0)
        pltpu.sync_copy(tut_hbm, sbt)
        n2 = process(TAIL64 - W30LO, sbt, u_lists, 64, n1, 1)
        drain_scatters(stage_a, gu_hbm, n2[0][0], ssem_a, cb2d_a)
        drain_scatters(stage_b, gu_hbm, n2[0][1], ssem_b, cb2d_b)

    @pl.when(wid != 30)
    def _():
        drain_scatters(stage_a, gu_hbm, ngs_u[0][0], ssem_a, cb2d_a)
        drain_scatters(stage_b, gu_hbm, ngs_u[0][1], ssem_b, cb2d_b)

    @pl.when(n_full > 0)
    def _():
        fire(it_hbm, 0, sb0, sem0)
        fire(it_hbm, 1, sb1, sem1)

    fill_p = extract(pidx_hbm, lrel_p, lb_p)
    fill_n = extract(nidx_hbm, lrel_n, lb_n)
    i_lists = [(lrel_p, lb_p, fill_p, gi_hbm, (stage_a, stage_b),
                (ssem_a, ssem_b), (cb2d_a, cb2d_b)),
               (lrel_n, lb_n, fill_n, gj_hbm, (stage_c, stage_d),
                (ssem_c, ssem_d), (cb2d_c, cb2d_d))]
    ngs_i = scan(it_hbm, i_lists, n_full, ((zero, zero), (zero, zero)))

    @pl.when(wid == 30)
    def _():
        pltpu.sync_copy(it_hbm.at[:, pl.ds(TAIL512 - W30LO + lo, 512)],
                        sb0.at[:, pl.ds(0, 512)])
        n1 = process(TAIL512 - W30LO, sb0, i_lists, 512, ngs_i, 0)
        pltpu.sync_copy(tit_hbm, sbt)
        n2 = process(TAIL64 - W30LO, sbt, i_lists, 64, n1, 1)
        drain_scatters(stage_a, gi_hbm, n2[0][0], ssem_a, cb2d_a)
        drain_scatters(stage_b, gi_hbm, n2[0][1], ssem_b, cb2d_b)
        drain_scatters(stage_c, gj_hbm, n2[1][0], ssem_c, cb2d_c)
        drain_scatters(stage_d, gj_hbm, n2[1][1], ssem_d, cb2d_d)

    @pl.when(wid != 30)
    def _():
        drain_scatters(stage_a, gi_hbm, ngs_i[0][0], ssem_a, cb2d_a)
        drain_scatters(stage_b, gi_hbm, ngs_i[0][1], ssem_b, cb2d_b)
        drain_scatters(stage_c, gj_hbm, ngs_i[1][0], ssem_c, cb2d_c)
        drain_scatters(stage_d, gj_hbm, ngs_i[1][1], ssem_d, cb2d_d)


def _dot_block(u_ref, i_ref, j_ref, p_ref, n_ref):
    u = u_ref[...]
    mask = lax.broadcasted_iota(jnp.int32, u.shape, 1) < DIM
    ui = jnp.where(mask, u * i_ref[...], 0.0)
    uj = jnp.where(mask, u * j_ref[...], 0.0)
    p_ref[...] = jnp.sum(ui, axis=1, keepdims=True)
    n_ref[...] = jnp.sum(uj, axis=1, keepdims=True)


@jax.jit
def _bpr(batch_user, batch_pos_item, batch_neg_item, user_emb, item_emb):
    u_t = user_emb.T
    i_t = item_emb.T
    t_u = lax.slice(u_t, (0, TAIL64), (DIM, USERS))
    t_i = lax.slice(i_t, (0, TAIL64), (DIM, USERS))
    mesh = plsc.VectorSubcoreMesh(core_axis_name="c", subcore_axis_name="s")
    kfn = pl.kernel(
        _scan_body,
        out_type=(
            jax.ShapeDtypeStruct((GROWS, WIDE), jnp.float32),
            jax.ShapeDtypeStruct((GROWS, WIDE), jnp.float32),
            jax.ShapeDtypeStruct((GROWS, WIDE), jnp.float32),
        ),
        mesh=mesh,
        compiler_params=pltpu.CompilerParams(needs_layout_passes=False),
        scratch_types=[
            pltpu.VMEM((BATCH,), jnp.int32),
            pltpu.VMEM((DIM, CHU), jnp.float32),
            pltpu.VMEM((DIM, CHU), jnp.float32),
            pltpu.VMEM((DIM, 64), jnp.float32),
            pltpu.VMEM((LCAP,), jnp.int32),
            pltpu.VMEM((LCAP,), jnp.int32),
            pltpu.VMEM((LCAP,), jnp.int32),
            pltpu.VMEM((LCAP,), jnp.int32),
            pltpu.VMEM((LCAP,), jnp.int32),
            pltpu.VMEM((LCAP,), jnp.int32),
            pltpu.VMEM((CCAP,), jnp.int32),
            pltpu.VMEM((CCAP // 16, 16), jnp.int32),
            pltpu.VMEM((CCAP // 16, 16), jnp.int32),
            pltpu.VMEM((CCAP // 16, 16), jnp.int32),
            pltpu.VMEM((CCAP // 16, 16), jnp.int32),
            pltpu.VMEM((CCAP, WIDE), jnp.float32),
            pltpu.VMEM((CCAP, WIDE), jnp.float32),
            pltpu.VMEM((CCAP, WIDE), jnp.float32),
            pltpu.VMEM((CCAP, WIDE), jnp.float32),
            pltpu.SemaphoreType.DMA,
            pltpu.SemaphoreType.DMA,
            pltpu.SemaphoreType.DMA,
            pltpu.SemaphoreType.DMA,
            pltpu.SemaphoreType.DMA,
            pltpu.SemaphoreType.DMA,
        ],
    )
    gu, gi, gj = kfn(batch_user, batch_pos_item, batch_neg_item, u_t, i_t,
                     t_u, t_i)

    grid = 8
    blk = BATCH // grid
    pos, neg = pl.pallas_call(
        _dot_block,
        grid=(grid,),
        in_specs=[
            pl.BlockSpec((blk, WIDE), lambda i: (i, 0)),
            pl.BlockSpec((blk, WIDE), lambda i: (i, 0)),
            pl.BlockSpec((blk, WIDE), lambda i: (i, 0)),
        ],
        out_specs=[
            pl.BlockSpec((blk, 1), lambda i: (i, 0)),
            pl.BlockSpec((blk, 1), lambda i: (i, 0)),
        ],
        out_shape=[
            jax.ShapeDtypeStruct((BATCH, 1), jnp.float32),
            jax.ShapeDtypeStruct((BATCH, 1), jnp.float32),
        ],
    )(gu, gi, gj)
    return pos, neg


def kernel(batch_user, batch_pos_item, batch_neg_item, user_emb, item_emb):
    return _bpr(batch_user, batch_pos_item, batch_neg_item,
                user_emb, item_emb)
